# Initial kernel scaffold; baseline (speedup 1.0000x reference)
import functools

import jax
import jax.numpy as jnp
from jax import lax
from jax.experimental import pallas as pl
from jax.experimental.pallas import tpu as pltpu
from jax.experimental.pallas import tpu_sc as plsc

N = 10000
D = 128
D_IN = 37
E = 320000
NRES = 1000

NC, NS, L = 2, 16, 16
NW = NC * NS
NPAD = 10240
ROWS_PER_SUB = NPAD // NS

CHUNK = 128
ECH = 80
EPT = ECH * CHUNK
EPAD = NW * EPT

NRESPAD = 1024
APT = NPAD // NS
ACHUNK = 64
ACH = APT // ACHUNK

_mesh = plsc.VectorSubcoreMesh(
    core_axis_name="c", subcore_axis_name="s", num_cores=NC, num_subcores=NS)


def _zero_vmem_2d(ref, rows):
    z = jnp.zeros((L,), jnp.float32)

    def body(r, _):
        for colb in range(D // L):
            ref[r, pl.ds(colb * L, L)] = z
        return 0

    lax.fori_loop(0, rows, body, 0)


def _zero_vmem_1d(ref, n):
    z = jnp.zeros((L,), jnp.float32)

    def body(i, _):
        ref[pl.ds(i * L, L)] = z
        return 0

    lax.fori_loop(0, n // L, body, 0)


def _agg_body(with_counts, x_hbm, sidx_hbm, didx_hbm, *rest):
    if with_counts:
        (out_s, out_cnt, acc, sidx, didx, rb0, rb1, zbuf, sem0, sem1,
         cntacc, ones, zb1) = rest
    else:
        out_s, acc, sidx, didx, rb0, rb1, zbuf, sem0, sem1 = rest

    c = lax.axis_index("c")
    s = lax.axis_index("s")
    w = c * NS + s
    base = s * ROWS_PER_SUB

    _zero_vmem_2d(zbuf, 64)

    def zacc(k, _):
        pltpu.sync_copy(zbuf, acc.at[pl.ds(base + k * 64, 64)])
        return 0

    lax.fori_loop(0, ROWS_PER_SUB // 64, zacc, 0)

    if with_counts:
        _zero_vmem_1d(zb1, ROWS_PER_SUB)
        pltpu.sync_copy(zb1, cntacc.at[pl.ds(base, ROWS_PER_SUB)])
        one = jnp.ones((L,), jnp.float32)

        def fill_ones(i, _):
            ones[pl.ds(i * L, L)] = one
            return 0

        lax.fori_loop(0, CHUNK // L, fill_ones, 0)

    pltpu.sync_copy(sidx_hbm.at[w], sidx)
    pltpu.sync_copy(didx_hbm.at[w], didx)

    plsc.subcore_barrier()

    pltpu.async_copy(x_hbm.at[sidx.at[0]], rb0, sem0)
    pltpu.async_copy(x_hbm.at[sidx.at[1]], rb1, sem1)

    def pair(jj, _):
        j0 = 2 * jj
        j1 = j0 + 1
        pltpu.make_async_copy(x_hbm.at[sidx.at[j0]], rb0, sem0).wait()
        pltpu.sync_copy(rb0, acc.at[didx.at[j0]], add=True)

        @pl.when(j0 + 2 < ECH)
        def _():
            pltpu.async_copy(x_hbm.at[sidx.at[j0 + 2]], rb0, sem0)

        pltpu.make_async_copy(x_hbm.at[sidx.at[j1]], rb1, sem1).wait()
        pltpu.sync_copy(rb1, acc.at[didx.at[j1]], add=True)

        @pl.when(j1 + 2 < ECH)
        def _():
            pltpu.async_copy(x_hbm.at[sidx.at[j1 + 2]], rb1, sem1)

        return 0

    lax.fori_loop(0, ECH // 2, pair, 0)

    if with_counts:
        def cnt_body(j, _):
            pltpu.sync_copy(ones, cntacc.at[didx.at[j]], add=True)
            return 0

        lax.fori_loop(0, ECH, cnt_body, 0)

    plsc.subcore_barrier()

    pltpu.sync_copy(acc.at[pl.ds(base, ROWS_PER_SUB)],
                    out_s.at[c, pl.ds(base, ROWS_PER_SUB)])
    if with_counts:
        pltpu.sync_copy(cntacc.at[pl.ds(base, ROWS_PER_SUB)],
                        out_cnt.at[c, pl.ds(base, ROWS_PER_SUB)])


def _make_agg(with_counts):
    out_type = [jax.ShapeDtypeStruct((NC, NPAD, D), jnp.float32)]
    scratch = [
        pltpu.VMEM_SHARED((NPAD, D), jnp.float32),
        pltpu.VMEM((ECH, CHUNK), jnp.int32),
        pltpu.VMEM((ECH, CHUNK), jnp.int32),
        pltpu.VMEM((CHUNK, D), jnp.float32),
        pltpu.VMEM((CHUNK, D), jnp.float32),
        pltpu.VMEM((64, D), jnp.float32),
        pltpu.SemaphoreType.DMA,
        pltpu.SemaphoreType.DMA,
    ]
    if with_counts:
        out_type.append(jax.ShapeDtypeStruct((NC, NPAD), jnp.float32))
        scratch += [
            pltpu.VMEM_SHARED((NPAD,), jnp.float32),
            pltpu.VMEM((CHUNK,), jnp.float32),
            pltpu.VMEM((ROWS_PER_SUB,), jnp.float32),
        ]
    return pl.kernel(
        functools.partial(_agg_body, with_counts),
        out_type=tuple(out_type),
        mesh=_mesh,
        scratch_types=scratch,
    )


_agg_with_counts = _make_agg(True)
_agg_plain = _make_agg(False)


def _res_body(x_hbm, aidx_hbm, out_hbm, resacc, abuf, aidx, zbuf):
    c = lax.axis_index("c")
    s = lax.axis_index("s")

    @pl.when(c == 0)
    def _():
        _zero_vmem_2d(zbuf, 64)
        pltpu.sync_copy(zbuf, resacc.at[pl.ds(s * 64, 64)])
        pltpu.sync_copy(aidx_hbm.at[s], aidx)
        pltpu.sync_copy(x_hbm.at[pl.ds(s * APT, APT)], abuf)

    plsc.subcore_barrier()

    @pl.when(c == 0)
    def _():
        def ch(j, _):
            pltpu.sync_copy(abuf.at[pl.ds(j * ACHUNK, ACHUNK)],
                            resacc.at[aidx.at[j]], add=True)
            return 0

        lax.fori_loop(0, ACH, ch, 0)

    plsc.subcore_barrier()

    @pl.when(c == 0)
    def _():
        pltpu.sync_copy(resacc.at[pl.ds(s * 64, 64)],
                        out_hbm.at[pl.ds(s * 64, 64)])


_res_scatter = pl.kernel(
    _res_body,
    out_type=jax.ShapeDtypeStruct((NRESPAD, D), jnp.float32),
    mesh=_mesh,
    scratch_types=[
        pltpu.VMEM_SHARED((NRESPAD, D), jnp.float32),
        pltpu.VMEM((APT, D), jnp.float32),
        pltpu.VMEM((ACH, ACHUNK), jnp.int32),
        pltpu.VMEM((64, D), jnp.float32),
    ],
)


_RB = 1024


def _linear_body(x_ref, wt_ref, b_ref, o_ref):
    o_ref[:] = (jnp.dot(x_ref[:], wt_ref[:], preferred_element_type=jnp.float32)
                + b_ref[:])


def _tc_linear(x, wt, b):
    return pl.pallas_call(
        _linear_body,
        out_shape=jax.ShapeDtypeStruct((NPAD, D), jnp.float32),
        grid=(NPAD // _RB,),
        in_specs=[
            pl.BlockSpec((_RB, D), lambda i: (i, 0)),
            pl.BlockSpec((D, D), lambda i: (0, 0)),
            pl.BlockSpec((1, D), lambda i: (0, 0)),
        ],
        out_specs=pl.BlockSpec((_RB, D), lambda i: (i, 0)),
    )(x, wt, b)


def _combine_body(s_ref, cnt_ref, x_ref, wlt_ref, wrt_ref, b_ref, o_ref):
    sm = s_ref[0] + s_ref[1]
    cnt = cnt_ref[0] + cnt_ref[1]
    agg = sm * (1.0 / jnp.maximum(cnt, 1.0))
    o_ref[:] = (jnp.dot(agg, wlt_ref[:], preferred_element_type=jnp.float32)
                + jnp.dot(x_ref[:], wrt_ref[:], preferred_element_type=jnp.float32)
                + b_ref[:])


def _tc_combine(s, cnt3, x, wlt, wrt, b):
    return pl.pallas_call(
        _combine_body,
        out_shape=jax.ShapeDtypeStruct((NPAD, D), jnp.float32),
        grid=(NPAD // _RB,),
        in_specs=[
            pl.BlockSpec((NC, _RB, D), lambda i: (0, i, 0)),
            pl.BlockSpec((NC, _RB, 1), lambda i: (0, i, 0)),
            pl.BlockSpec((_RB, D), lambda i: (i, 0)),
            pl.BlockSpec((D, D), lambda i: (0, 0)),
            pl.BlockSpec((D, D), lambda i: (0, 0)),
            pl.BlockSpec((1, D), lambda i: (0, 0)),
        ],
        out_specs=pl.BlockSpec((_RB, D), lambda i: (i, 0)),
    )(s, cnt3, x, wlt, wrt, b)


def kernel(resid_embedding, atom_onehot_embedding, edge_index_atom, res2atom_map,
           ln_W, ln_b, W1l, W1r, b1, W2l, W2r, b2, W3l, W3r, b3, W4l, W4r, b4):
    f32 = jnp.float32

    xin = jnp.zeros((NPAD, D), f32).at[:N, :D_IN].set(atom_onehot_embedding)
    lnWt = jnp.zeros((D, D), f32).at[:D_IN].set(ln_W.T)

    src = edge_index_atom[0].astype(jnp.int32)
    dst = edge_index_atom[1].astype(jnp.int32)
    pad_e = EPAD - E
    sidx = jnp.concatenate([src, jnp.zeros((pad_e,), jnp.int32)])
    didx = jnp.concatenate([dst, jnp.full((pad_e,), N, jnp.int32)])
    sidx = sidx.reshape(NW, ECH, CHUNK)
    didx = didx.reshape(NW, ECH, CHUNK)

    r2a = jnp.concatenate([
        res2atom_map.astype(jnp.int32),
        jnp.full((NPAD - N,), NRES, jnp.int32),
    ]).reshape(NS, ACH, ACHUNK)

    x = _tc_linear(xin, lnWt, ln_b.reshape(1, D))

    s, cnt = _agg_with_counts(x, sidx, didx)
    cnt3 = cnt.reshape(NC, NPAD, 1)
    x = _tc_combine(s, cnt3, x, W1l.T, W1r.T, b1.reshape(1, D))

    for Wl, Wr, b in ((W2l, W2r, b2), (W3l, W3r, b3), (W4l, W4r, b4)):
        (s,) = _agg_plain(x, sidx, didx)
        x = _tc_combine(s, cnt3, x, Wl.T, Wr.T, b.reshape(1, D))

    res = _res_scatter(x, r2a)
    return res[:NRES]


# speedup vs baseline: 3.2296x; 3.2296x over previous
import functools

import jax
import jax.numpy as jnp
from jax import lax
from jax.experimental import pallas as pl
from jax.experimental.pallas import tpu as pltpu
from jax.experimental.pallas import tpu_sc as plsc

N = 10000
D = 128
D_IN = 37
E = 320000
NRES = 1000

NC, NS, L = 2, 16, 16
NW = NC * NS
NPAD = 10240
ROWS_PER_SUB = NPAD // NS

CHUNK = 128
ECH = 80
KBLK = 8
EPT = ECH * CHUNK
EPAD = NW * EPT

NRESPAD = 1024
APT = NPAD // NS
ACHUNK = 128
ACH = APT // ACHUNK

_mesh = plsc.VectorSubcoreMesh(
    core_axis_name="c", subcore_axis_name="s", num_cores=NC, num_subcores=NS)


def _zero_vmem_2d(ref, rows):
    z = jnp.zeros((L,), jnp.float32)

    def body(r, _):
        for colb in range(D // L):
            ref[r, pl.ds(colb * L, L)] = z
        return 0

    lax.fori_loop(0, rows, body, 0)


def _zero_vmem_1d(ref, n):
    z = jnp.zeros((L,), jnp.float32)

    def body(i, _):
        ref[pl.ds(i * L, L)] = z
        return 0

    lax.fori_loop(0, n // L, body, 0)


def _agg_body(with_counts, x_hbm, sidx_hbm, didx_hbm, *rest):
    if with_counts:
        (out_s, out_cnt, acc, sblk, dblk, rb, sem0, sem1,
         cntacc, ones, zb1) = rest
    else:
        out_s, acc, sblk, dblk, rb, sem0, sem1 = rest

    c = lax.axis_index("c")
    s = lax.axis_index("s")
    w = c * NS + s
    base = s * ROWS_PER_SUB

    z = jnp.zeros((L,), jnp.float32)

    def zrow(r, _):
        for colb in range(D // L):
            rb[0, r, pl.ds(colb * L, L)] = z
        return 0

    lax.fori_loop(0, CHUNK, zrow, 0)

    def zacc(k, _):
        pltpu.sync_copy(rb.at[0], acc.at[pl.ds(base + k * CHUNK, CHUNK)])
        return 0

    lax.fori_loop(0, ROWS_PER_SUB // CHUNK, zacc, 0)

    if with_counts:
        _zero_vmem_1d(zb1, ROWS_PER_SUB)
        pltpu.sync_copy(zb1, cntacc.at[pl.ds(base, ROWS_PER_SUB)])
        one = jnp.ones((L,), jnp.float32)

        def fill_ones(i, _):
            ones[pl.ds(i * L, L)] = one
            return 0

        lax.fori_loop(0, CHUNK // L, fill_ones, 0)

    plsc.subcore_barrier()

    sems = (sem0, sem1)

    def blk(b, _):
        pltpu.sync_copy(sidx_hbm.at[w, pl.ds(b * KBLK, KBLK)], sblk)
        pltpu.sync_copy(didx_hbm.at[w, pl.ds(b * KBLK, KBLK)], dblk)
        pltpu.async_copy(x_hbm.at[sblk.at[0]], rb.at[0], sems[0])
        for k in range(KBLK):
            cur = k % 2
            if k + 1 < KBLK:
                nxt = (k + 1) % 2
                pltpu.async_copy(x_hbm.at[sblk.at[k + 1]], rb.at[nxt], sems[nxt])
            pltpu.make_async_copy(x_hbm.at[sblk.at[k]], rb.at[cur],
                                  sems[cur]).wait()
            pltpu.sync_copy(rb.at[cur], acc.at[dblk.at[k]], add=True)
            if with_counts:
                pltpu.sync_copy(ones, cntacc.at[dblk.at[k]], add=True)
        return 0

    lax.fori_loop(0, ECH // KBLK, blk, 0)

    plsc.subcore_barrier()

    pltpu.sync_copy(acc.at[pl.ds(base, ROWS_PER_SUB)],
                    out_s.at[c, pl.ds(base, ROWS_PER_SUB)])
    if with_counts:
        pltpu.sync_copy(cntacc.at[pl.ds(base, ROWS_PER_SUB)],
                        out_cnt.at[c, pl.ds(base, ROWS_PER_SUB)])


def _make_agg(with_counts):
    out_type = [jax.ShapeDtypeStruct((NC, NPAD, D), jnp.float32)]
    scratch = [
        pltpu.VMEM_SHARED((NPAD, D), jnp.float32),
        pltpu.VMEM((KBLK, CHUNK), jnp.int32),
        pltpu.VMEM((KBLK, CHUNK), jnp.int32),
        pltpu.VMEM((2, CHUNK, D), jnp.float32),
        pltpu.SemaphoreType.DMA,
        pltpu.SemaphoreType.DMA,
    ]
    if with_counts:
        out_type.append(jax.ShapeDtypeStruct((NC, NPAD), jnp.float32))
        scratch += [
            pltpu.VMEM_SHARED((NPAD,), jnp.float32),
            pltpu.VMEM((CHUNK,), jnp.float32),
            pltpu.VMEM((ROWS_PER_SUB,), jnp.float32),
        ]
    return pl.kernel(
        functools.partial(_agg_body, with_counts),
        out_type=tuple(out_type),
        mesh=_mesh,
        scratch_types=scratch,
    )


_agg_with_counts = _make_agg(True)
_agg_plain = _make_agg(False)


def _res_body(x_hbm, aidx_hbm, out_hbm, resacc, abuf, aidx, zbuf):
    c = lax.axis_index("c")
    s = lax.axis_index("s")

    @pl.when(c == 0)
    def _():
        _zero_vmem_2d(zbuf, 64)
        pltpu.sync_copy(zbuf, resacc.at[pl.ds(s * 64, 64)])
        pltpu.sync_copy(aidx_hbm.at[s], aidx)
        pltpu.sync_copy(x_hbm.at[pl.ds(s * APT, APT)], abuf)

    plsc.subcore_barrier()

    @pl.when(c == 0)
    def _():
        def ch(j, _):
            pltpu.sync_copy(abuf.at[pl.ds(j * ACHUNK, ACHUNK)],
                            resacc.at[aidx.at[j]], add=True)
            return 0

        lax.fori_loop(0, ACH, ch, 0)

    plsc.subcore_barrier()

    @pl.when(c == 0)
    def _():
        pltpu.sync_copy(resacc.at[pl.ds(s * 64, 64)],
                        out_hbm.at[pl.ds(s * 64, 64)])


_res_scatter = pl.kernel(
    _res_body,
    out_type=jax.ShapeDtypeStruct((NRESPAD, D), jnp.float32),
    mesh=_mesh,
    scratch_types=[
        pltpu.VMEM_SHARED((NRESPAD, D), jnp.float32),
        pltpu.VMEM((APT, D), jnp.float32),
        pltpu.VMEM((ACH, ACHUNK), jnp.int32),
        pltpu.VMEM((64, D), jnp.float32),
    ],
)


_RB = 1024


def _linear_body(x_ref, wt_ref, b_ref, o_ref):
    o_ref[:] = (jnp.dot(x_ref[:], wt_ref[:], preferred_element_type=jnp.float32)
                + b_ref[:])


def _tc_linear(x, wt, b):
    return pl.pallas_call(
        _linear_body,
        out_shape=jax.ShapeDtypeStruct((NPAD, D), jnp.float32),
        grid=(NPAD // _RB,),
        in_specs=[
            pl.BlockSpec((_RB, D), lambda i: (i, 0)),
            pl.BlockSpec((D, D), lambda i: (0, 0)),
            pl.BlockSpec((1, D), lambda i: (0, 0)),
        ],
        out_specs=pl.BlockSpec((_RB, D), lambda i: (i, 0)),
    )(x, wt, b)


def _combine_body(s_ref, cnt_ref, x_ref, wlt_ref, wrt_ref, b_ref, o_ref):
    sm = s_ref[0] + s_ref[1]
    cnt = cnt_ref[0] + cnt_ref[1]
    agg = sm * (1.0 / jnp.maximum(cnt, 1.0))
    o_ref[:] = (jnp.dot(agg, wlt_ref[:], preferred_element_type=jnp.float32)
                + jnp.dot(x_ref[:], wrt_ref[:], preferred_element_type=jnp.float32)
                + b_ref[:])


def _tc_combine(s, cnt3, x, wlt, wrt, b):
    return pl.pallas_call(
        _combine_body,
        out_shape=jax.ShapeDtypeStruct((NPAD, D), jnp.float32),
        grid=(NPAD // _RB,),
        in_specs=[
            pl.BlockSpec((NC, _RB, D), lambda i: (0, i, 0)),
            pl.BlockSpec((NC, _RB, 1), lambda i: (0, i, 0)),
            pl.BlockSpec((_RB, D), lambda i: (i, 0)),
            pl.BlockSpec((D, D), lambda i: (0, 0)),
            pl.BlockSpec((D, D), lambda i: (0, 0)),
            pl.BlockSpec((1, D), lambda i: (0, 0)),
        ],
        out_specs=pl.BlockSpec((_RB, D), lambda i: (i, 0)),
    )(s, cnt3, x, wlt, wrt, b)


def kernel(resid_embedding, atom_onehot_embedding, edge_index_atom, res2atom_map,
           ln_W, ln_b, W1l, W1r, b1, W2l, W2r, b2, W3l, W3r, b3, W4l, W4r, b4):
    f32 = jnp.float32

    xin = jnp.zeros((NPAD, D), f32).at[:N, :D_IN].set(atom_onehot_embedding)
    lnWt = jnp.zeros((D, D), f32).at[:D_IN].set(ln_W.T)

    src = edge_index_atom[0].astype(jnp.int32)
    dst = edge_index_atom[1].astype(jnp.int32)
    pad_e = EPAD - E
    sidx = jnp.concatenate([src, jnp.zeros((pad_e,), jnp.int32)])
    didx = jnp.concatenate([dst, jnp.full((pad_e,), N, jnp.int32)])
    sidx = sidx.reshape(NW, ECH, CHUNK)
    didx = didx.reshape(NW, ECH, CHUNK)

    r2a = jnp.concatenate([
        res2atom_map.astype(jnp.int32),
        jnp.full((NPAD - N,), NRES, jnp.int32),
    ]).reshape(NS, ACH, ACHUNK)

    x = _tc_linear(xin, lnWt, ln_b.reshape(1, D))

    s, cnt = _agg_with_counts(x, sidx, didx)
    cnt3 = cnt.reshape(NC, NPAD, 1)
    x = _tc_combine(s, cnt3, x, W1l.T, W1r.T, b1.reshape(1, D))

    for Wl, Wr, b in ((W2l, W2r, b2), (W3l, W3r, b3), (W4l, W4r, b4)):
        (s,) = _agg_plain(x, sidx, didx)
        x = _tc_combine(s, cnt3, x, Wl.T, Wr.T, b.reshape(1, D))

    res = _res_scatter(x, r2a)
    return res[:NRES]
